# baseline (speedup 1.0000x reference)
import jax
import jax.numpy as jnp
from jax import lax
from jax.experimental import pallas as pl
from jax.experimental.pallas import tpu as pltpu
from jax.experimental.pallas import tpu_sc as plsc

N = 10000
D = 128
E = 320000
NC = 2
NS = 16
NW = NC * NS
K = 128
N_PAD = 10240
E_PAD = 327680
RPT = N_PAD // NS
NBLK_FAST = 120
NBLK_SLOW = 40
FAST_CORE = 1
NBLK_DEG = (E_PAD // K) // NW
EPT = NBLK_FAST * K
E_ALLOC = NS * (NBLK_FAST + NBLK_SLOW) * K + (EPT - NBLK_SLOW * K)

_F32 = jnp.float32


def _sc_mesh():
    return plsc.VectorSubcoreMesh(
        core_axis_name="c", subcore_axis_name="s",
        num_cores=NC, num_subcores=NS)


def _cvt_block(srcpk, poff, dst32):
    poff = pl.multiple_of(poff, 8)
    for q in range(K // 32):
        v = srcpk[pl.ds(poff + q * 16, 16)]
        dst32[pl.ds(q * 16, 16)] = v & 0xFFFF
        dst32[pl.ds(K // 2 + q * 16, 16)] = v >> 16


def _deg_body(col_hbm, out_hbm, col16, colv, onesv, zv, acc):
    c = lax.axis_index("c")
    s = lax.axis_index("s")
    wid = c * NS + s
    pltpu.sync_copy(col_hbm.at[pl.ds(pl.multiple_of(wid * (NBLK_DEG * K // 2),
                                                    8),
                                     NBLK_DEG * K // 2)], col16)
    for j in range(K // 16):
        onesv[pl.ds(j * 16, 16)] = jnp.ones((16,), _F32)

    def zfill(j, carry):
        zv[pl.ds(j * 16, 16)] = jnp.zeros((16,), _F32)
        return carry

    lax.fori_loop(0, RPT // 16, zfill, 0)
    pltpu.sync_copy(zv, acc.at[pl.ds(s * RPT, RPT)])
    plsc.subcore_barrier()

    def step(i, carry):
        _cvt_block(col16, i * (K // 2), colv)
        pltpu.sync_copy(onesv, acc.at[colv], add=True)
        return carry

    lax.fori_loop(0, NBLK_DEG, step, 0)
    plsc.subcore_barrier()
    pltpu.sync_copy(acc.at[pl.ds(s * RPT, RPT)],
                    out_hbm.at[c, pl.ds(s * RPT, RPT)])


NBUF = 2


def _agg_body(table_hbm, row_hbm, col_hbm, out_hbm, *refs):
    row16 = refs[0]
    col16 = refs[1]
    rowv = refs[2:2 + NBUF]
    colv = refs[2 + NBUF:2 + 2 * NBUF]
    msgs = refs[2 + 2 * NBUF:2 + 3 * NBUF]
    acc = refs[2 + 3 * NBUF]
    gsem = refs[3 + 3 * NBUF:3 + 4 * NBUF]
    ssem = refs[3 + 4 * NBUF:3 + 5 * NBUF]
    c = lax.axis_index("c")
    s = lax.axis_index("s")

    fast = c == FAST_CORE
    base0 = jnp.where(fast, s * EPT, NS * EPT + s * NBLK_SLOW * K) // 2
    base0 = pl.multiple_of(base0, 8)
    nblk = jnp.where(fast, NBLK_FAST, NBLK_SLOW)

    pltpu.sync_copy(row_hbm.at[pl.ds(base0, EPT // 2)], row16)
    pltpu.sync_copy(col_hbm.at[pl.ds(base0, EPT // 2)], col16)

    def zrow(i, carry):
        for j in range(D // 16):
            msgs[0][i, pl.ds(j * 16, 16)] = jnp.zeros((16,), _F32)
        return carry

    lax.fori_loop(0, K, zrow, 0)
    for b in range(RPT // K):
        pltpu.sync_copy(msgs[0], acc.at[pl.ds(s * RPT + b * K, K)])
    plsc.subcore_barrier()

    def fetch(p, j):
        _cvt_block(row16, j * (K // 2), rowv[p])
        _cvt_block(col16, j * (K // 2), colv[p])
        pltpu.async_copy(table_hbm.at[rowv[p]], msgs[p], gsem[p])

    def start_scatter(p):
        pltpu.make_async_copy(table_hbm.at[rowv[p]], msgs[p], gsem[p]).wait()
        pltpu.async_copy(msgs[p], acc.at[colv[p]], ssem[p], add=True)

    def wait_scatter(p):
        pltpu.make_async_copy(msgs[p], acc.at[colv[p]], ssem[p]).wait()

    for p in range(NBUF):
        fetch(p, p)

    def step(i, carry):
        j = i * NBUF
        for p in range(NBUF):
            start_scatter(p)
            wait_scatter(p)
            fetch(p, j + NBUF + p)
        return carry

    lax.fori_loop(0, nblk // NBUF - 1, step, 0)
    for p in range(NBUF):
        start_scatter(p)
        wait_scatter(p)
    plsc.subcore_barrier()
    for b in range(RPT // K):
        r0 = s * RPT + b * K
        pltpu.sync_copy(acc.at[pl.ds(r0, K)], out_hbm.at[c, pl.ds(r0, K)])


def _scale_body(deg_ref, x_ref, dinv_ref, xs_ref):
    d = deg_ref[...]
    dinv = jnp.where(d > 0.0, lax.rsqrt(d), 0.0)
    dinv_ref[...] = dinv
    xs_ref[...] = x_ref[...] * dinv


def _mid_body(acc_ref, dinv_ref, w_ref, b_ref, g_ref, bt_ref, hs_ref):
    dinv = dinv_ref[...]
    agg = (acc_ref[0] + acc_ref[1]) * dinv
    h = jnp.dot(agg, w_ref[...], preferred_element_type=_F32,
                precision=lax.Precision.HIGHEST) + b_ref[...]
    hv = h[:N]
    mean = jnp.mean(hv, axis=0, keepdims=True)
    var = jnp.mean((hv - mean) ** 2, axis=0, keepdims=True)
    hn = (h - mean) * lax.rsqrt(var + 1e-5) * g_ref[...] + bt_ref[...]
    hs_ref[...] = jnp.maximum(hn, 0.0) * dinv


def _head_body(acc_ref, dinv_ref, w2_ref, b2_ref, w3_ref, b3_ref,
               f_ref, o_ref):
    agg = (acc_ref[0] + acc_ref[1]) * dinv_ref[...]
    f_ref[...] = jnp.dot(agg, w2_ref[...], preferred_element_type=_F32,
                         precision=lax.Precision.HIGHEST) + b2_ref[...]
    o_ref[...] = jnp.dot(agg, w3_ref[...], preferred_element_type=_F32,
                         precision=lax.Precision.HIGHEST) + b3_ref[...]


def kernel(x, edge_index, W1, b1, gamma, beta, W2, b2, W3, b3):
    row = edge_index[0]
    col = edge_index[1]
    pad_e = jnp.full((E_ALLOC - E,), N, jnp.int32)
    rp = jnp.concatenate([row, pad_e]).reshape(-1, 2)
    cp = jnp.concatenate([col, pad_e]).reshape(-1, 2)
    rowp = rp[:, 0] | (rp[:, 1] << 16)
    colp = cp[:, 0] | (cp[:, 1] << 16)
    x_pad = jnp.concatenate([x, jnp.zeros((N_PAD - N, D), _F32)], axis=0)

    deg_call = pl.kernel(
        _deg_body,
        out_type=jax.ShapeDtypeStruct((NC, N_PAD), _F32),
        mesh=_sc_mesh(),
        scratch_types=[
            pltpu.VMEM((NBLK_DEG * K // 2,), jnp.int32),
            pltpu.VMEM((K,), jnp.int32),
            pltpu.VMEM((K,), _F32),
            pltpu.VMEM((RPT,), _F32),
            pltpu.VMEM_SHARED((N_PAD,), _F32),
        ])
    agg_call = pl.kernel(
        _agg_body,
        out_type=jax.ShapeDtypeStruct((NC, N_PAD, D), _F32),
        mesh=_sc_mesh(),
        scratch_types=(
            [pltpu.VMEM((EPT // 2,), jnp.int32) for _ in range(2)]
            + [pltpu.VMEM((K,), jnp.int32) for _ in range(2 * NBUF)]
            + [pltpu.VMEM((K, D), _F32) for _ in range(NBUF)]
            + [pltpu.VMEM_SHARED((N_PAD, D), _F32)]
            + [pltpu.SemaphoreType.DMA for _ in range(2 * NBUF)]
        ))

    scale_call = pl.pallas_call(
        _scale_body,
        out_shape=(jax.ShapeDtypeStruct((N_PAD, 1), _F32),
                   jax.ShapeDtypeStruct((N_PAD, D), _F32)))
    mid_call = pl.pallas_call(
        _mid_body,
        out_shape=jax.ShapeDtypeStruct((N_PAD, D), _F32))
    head_call = pl.pallas_call(
        _head_body,
        out_shape=(jax.ShapeDtypeStruct((N_PAD, D), _F32),
                   jax.ShapeDtypeStruct((N_PAD, D), _F32)))

    deg2 = deg_call(colp)
    deg_col = (deg2[0] + deg2[1]).reshape(N_PAD, 1)
    dinv_col, xs = scale_call(deg_col, x_pad)
    acc1 = agg_call(xs, rowp, colp)
    hs = mid_call(acc1, dinv_col, W1, b1.reshape(1, D), gamma.reshape(1, D),
                  beta.reshape(1, D))
    acc2 = agg_call(hs, rowp, colp)
    features, out = head_call(acc2, dinv_col, W2, b2.reshape(1, D), W3,
                              b3.reshape(1, D))
    return features[:N], out[:N]


# speedup vs baseline: 1.4086x; 1.4086x over previous
import jax
import jax.numpy as jnp
from jax import lax
from jax.experimental import pallas as pl
from jax.experimental.pallas import tpu as pltpu
from jax.experimental.pallas import tpu_sc as plsc

N = 10000
D = 128
E = 320000
NC = 2
NS = 16
NW = NC * NS
K = 128
N_PAD = 10240
E_PAD = 327680
RPT = N_PAD // NS
NBLK_FAST = 120
NBLK_SLOW = 40
FAST_CORE = 0
NBLK_DEG = (E_PAD // K) // NW
EPT = NBLK_FAST * K

_F32 = jnp.float32


def _sc_mesh():
    return plsc.VectorSubcoreMesh(
        core_axis_name="c", subcore_axis_name="s",
        num_cores=NC, num_subcores=NS)


def _deg_body(col_hbm, out_hbm, colv, onesv, zv, acc):
    c = lax.axis_index("c")
    s = lax.axis_index("s")
    wid = c * NS + s
    for j in range(K // 16):
        onesv[pl.ds(j * 16, 16)] = jnp.ones((16,), _F32)

    def zfill(j, carry):
        zv[pl.ds(j * 16, 16)] = jnp.zeros((16,), _F32)
        return carry

    lax.fori_loop(0, RPT // 16, zfill, 0)
    pltpu.sync_copy(zv, acc.at[pl.ds(s * RPT, RPT)])
    plsc.subcore_barrier()

    def step(i, carry):
        pltpu.sync_copy(col_hbm.at[pl.ds((wid * NBLK_DEG + i) * K, K)], colv)
        pltpu.sync_copy(onesv, acc.at[colv], add=True)
        return carry

    lax.fori_loop(0, NBLK_DEG, step, 0)
    plsc.subcore_barrier()
    pltpu.sync_copy(acc.at[pl.ds(s * RPT, RPT)],
                    out_hbm.at[c, pl.ds(s * RPT, RPT)])


NBUF = 2


def _agg_body(table_hbm, row_hbm, col_hbm, out_hbm, *refs):
    rowv = refs[0:NBUF]
    colv = refs[NBUF:2 * NBUF]
    msgs = refs[2 * NBUF:3 * NBUF]
    acc = refs[3 * NBUF]
    gsem = refs[3 * NBUF + 1:3 * NBUF + 1 + NBUF]
    c = lax.axis_index("c")
    s = lax.axis_index("s")

    fast = c == FAST_CORE
    base0 = jnp.where(fast, s * EPT, NS * EPT + s * NBLK_SLOW * K)
    base0 = pl.multiple_of(base0, 8)
    nblk = jnp.where(fast, NBLK_FAST, NBLK_SLOW)

    def zrow(i, carry):
        for j in range(D // 16):
            msgs[0][i, pl.ds(j * 16, 16)] = jnp.zeros((16,), _F32)
        return carry

    lax.fori_loop(0, K, zrow, 0)
    for b in range(RPT // K):
        pltpu.sync_copy(msgs[0], acc.at[pl.ds(s * RPT + b * K, K)])
    plsc.subcore_barrier()

    def fetch(p, j):
        base = base0 + j * K
        pltpu.sync_copy(row_hbm.at[pl.ds(base, K)], rowv[p])
        pltpu.sync_copy(col_hbm.at[pl.ds(base, K)], colv[p])
        pltpu.async_copy(table_hbm.at[rowv[p]], msgs[p], gsem[p])

    def drain_scatter(p):
        pltpu.make_async_copy(table_hbm.at[rowv[p]], msgs[p], gsem[p]).wait()
        pltpu.sync_copy(msgs[p], acc.at[colv[p]], add=True)

    for p in range(NBUF):
        fetch(p, p)

    def step(i, carry):
        j = i * NBUF
        for p in range(NBUF):
            drain_scatter(p)
            fetch(p, j + NBUF + p)
        return carry

    lax.fori_loop(0, nblk // NBUF - 1, step, 0)
    for p in range(NBUF):
        drain_scatter(p)
    plsc.subcore_barrier()
    for b in range(RPT // K):
        r0 = s * RPT + b * K
        pltpu.sync_copy(acc.at[pl.ds(r0, K)], out_hbm.at[c, pl.ds(r0, K)])


def _scale_body(deg_ref, x_ref, dinv_ref, xs_ref):
    d = deg_ref[...]
    dinv = jnp.where(d > 0.0, lax.rsqrt(d), 0.0)
    dinv_ref[...] = dinv
    xs_ref[...] = x_ref[...] * dinv


def _mid_body(acc_ref, dinv_ref, w_ref, b_ref, g_ref, bt_ref, hs_ref):
    dinv = dinv_ref[...]
    agg = (acc_ref[0] + acc_ref[1]) * dinv
    h = jnp.dot(agg, w_ref[...], preferred_element_type=_F32,
                precision=lax.Precision.HIGHEST) + b_ref[...]
    hv = h[:N]
    mean = jnp.mean(hv, axis=0, keepdims=True)
    var = jnp.mean((hv - mean) ** 2, axis=0, keepdims=True)
    hn = (h - mean) * lax.rsqrt(var + 1e-5) * g_ref[...] + bt_ref[...]
    hs_ref[...] = jnp.maximum(hn, 0.0) * dinv


def _head_body(acc_ref, dinv_ref, w2_ref, b2_ref, w3_ref, b3_ref,
               f_ref, o_ref):
    agg = (acc_ref[0] + acc_ref[1]) * dinv_ref[...]
    f_ref[...] = jnp.dot(agg, w2_ref[...], preferred_element_type=_F32,
                         precision=lax.Precision.HIGHEST) + b2_ref[...]
    o_ref[...] = jnp.dot(agg, w3_ref[...], preferred_element_type=_F32,
                         precision=lax.Precision.HIGHEST) + b3_ref[...]


def kernel(x, edge_index, W1, b1, gamma, beta, W2, b2, W3, b3):
    row = edge_index[0]
    col = edge_index[1]
    pad_e = jnp.full((E_PAD - E,), N, jnp.int32)
    rowp = jnp.concatenate([row, pad_e])
    colp = jnp.concatenate([col, pad_e])
    x_pad = jnp.concatenate([x, jnp.zeros((N_PAD - N, D), _F32)], axis=0)

    deg_call = pl.kernel(
        _deg_body,
        out_type=jax.ShapeDtypeStruct((NC, N_PAD), _F32),
        mesh=_sc_mesh(),
        scratch_types=[
            pltpu.VMEM((K,), jnp.int32),
            pltpu.VMEM((K,), _F32),
            pltpu.VMEM((RPT,), _F32),
            pltpu.VMEM_SHARED((N_PAD,), _F32),
        ])
    agg_call = pl.kernel(
        _agg_body,
        out_type=jax.ShapeDtypeStruct((NC, N_PAD, D), _F32),
        mesh=_sc_mesh(),
        scratch_types=(
            [pltpu.VMEM((K,), jnp.int32) for _ in range(2 * NBUF)]
            + [pltpu.VMEM((K, D), _F32) for _ in range(NBUF)]
            + [pltpu.VMEM_SHARED((N_PAD, D), _F32)]
            + [pltpu.SemaphoreType.DMA for _ in range(NBUF)]
        ))

    scale_call = pl.pallas_call(
        _scale_body,
        out_shape=(jax.ShapeDtypeStruct((N_PAD, 1), _F32),
                   jax.ShapeDtypeStruct((N_PAD, D), _F32)))
    mid_call = pl.pallas_call(
        _mid_body,
        out_shape=jax.ShapeDtypeStruct((N_PAD, D), _F32))
    head_call = pl.pallas_call(
        _head_body,
        out_shape=(jax.ShapeDtypeStruct((N_PAD, D), _F32),
                   jax.ShapeDtypeStruct((N_PAD, D), _F32)))

    deg2 = deg_call(colp)
    deg_col = (deg2[0] + deg2[1]).reshape(N_PAD, 1)
    dinv_col, xs = scale_call(deg_col, x_pad)
    acc1 = agg_call(xs, rowp, colp)
    hs = mid_call(acc1, dinv_col, W1, b1.reshape(1, D), gamma.reshape(1, D),
                  beta.reshape(1, D))
    acc2 = agg_call(hs, rowp, colp)
    features, out = head_call(acc2, dinv_col, W2, b2.reshape(1, D), W3,
                              b3.reshape(1, D))
    return features[:N], out[:N]


# speedup vs baseline: 1.4096x; 1.0007x over previous
import jax
import jax.numpy as jnp
from jax import lax
from jax.experimental import pallas as pl
from jax.experimental.pallas import tpu as pltpu
from jax.experimental.pallas import tpu_sc as plsc

N = 10000
D = 128
E = 320000
NC = 2
NS = 16
NW = NC * NS
K = 128
N_PAD = 10240
E_PAD = 327680
RPT = N_PAD // NS
NBLK_FAST = 120
NBLK_SLOW = 40
FAST_CORE = 0
NBLK_DEG = (E_PAD // K) // NW
EPT = NBLK_FAST * K

_F32 = jnp.float32


def _sc_mesh():
    return plsc.VectorSubcoreMesh(
        core_axis_name="c", subcore_axis_name="s",
        num_cores=NC, num_subcores=NS)


def _deg_body(col_hbm, out_hbm, colv, onesv, zv, acc):
    c = lax.axis_index("c")
    s = lax.axis_index("s")
    wid = c * NS + s
    for j in range(K // 16):
        onesv[pl.ds(j * 16, 16)] = jnp.ones((16,), _F32)

    def zfill(j, carry):
        zv[pl.ds(j * 16, 16)] = jnp.zeros((16,), _F32)
        return carry

    lax.fori_loop(0, RPT // 16, zfill, 0)
    pltpu.sync_copy(zv, acc.at[pl.ds(s * RPT, RPT)])
    plsc.subcore_barrier()

    def step(i, carry):
        pltpu.sync_copy(col_hbm.at[pl.ds((wid * NBLK_DEG + i) * K, K)], colv)
        pltpu.sync_copy(onesv, acc.at[colv], add=True)
        return carry

    lax.fori_loop(0, NBLK_DEG, step, 0)
    plsc.subcore_barrier()
    pltpu.sync_copy(acc.at[pl.ds(s * RPT, RPT)],
                    out_hbm.at[c, pl.ds(s * RPT, RPT)])


NBUF = 2


def _agg_body(table_hbm, row_hbm, col_hbm, out_hbm, *refs):
    rowv = refs[0:NBUF]
    colv = refs[NBUF:2 * NBUF]
    msgs = refs[2 * NBUF:3 * NBUF]
    acc = refs[3 * NBUF]
    gsem = refs[3 * NBUF + 1:3 * NBUF + 1 + NBUF]
    c = lax.axis_index("c")
    s = lax.axis_index("s")

    fast = c == FAST_CORE
    base0 = jnp.where(fast, s * EPT, NS * EPT + s * NBLK_SLOW * K)
    base0 = pl.multiple_of(base0, 8)
    nblk = jnp.where(fast, NBLK_FAST, NBLK_SLOW)

    with jax.named_scope("agg_zero"):
        def zrow(i, carry):
            for j in range(D // 16):
                msgs[0][i, pl.ds(j * 16, 16)] = jnp.zeros((16,), _F32)
            return carry

        lax.fori_loop(0, K, zrow, 0)
        for b in range(RPT // K):
            pltpu.sync_copy(msgs[0], acc.at[pl.ds(s * RPT + b * K, K)])
        plsc.subcore_barrier()

    def fetch(p, j):
        base = base0 + j * K
        pltpu.sync_copy(row_hbm.at[pl.ds(base, K)], rowv[p])
        pltpu.sync_copy(col_hbm.at[pl.ds(base, K)], colv[p])
        pltpu.async_copy(table_hbm.at[rowv[p]], msgs[p], gsem[p])

    def drain_scatter(p):
        pltpu.make_async_copy(table_hbm.at[rowv[p]], msgs[p], gsem[p]).wait()
        pltpu.sync_copy(msgs[p], acc.at[colv[p]], add=True)

    with jax.named_scope("agg_edges"):
        for p in range(NBUF):
            fetch(p, p)

        def step(i, carry):
            j = i * NBUF
            for p in range(NBUF):
                drain_scatter(p)
                fetch(p, j + NBUF + p)
            return carry

        lax.fori_loop(0, nblk // NBUF - 1, step, 0)
        for p in range(NBUF):
            drain_scatter(p)
        plsc.subcore_barrier()
    with jax.named_scope("agg_writeout"):
        for b in range(RPT // K):
            r0 = s * RPT + b * K
            pltpu.sync_copy(acc.at[pl.ds(r0, K)], out_hbm.at[c, pl.ds(r0, K)])


def _scale_body(deg_ref, x_ref, dinv_ref, xs_ref):
    d = deg_ref[...]
    dinv = jnp.where(d > 0.0, lax.rsqrt(d), 0.0)
    dinv_ref[...] = dinv
    xs_ref[...] = x_ref[...] * dinv


def _mid_body(acc_ref, dinv_ref, w_ref, b_ref, g_ref, bt_ref, hs_ref):
    dinv = dinv_ref[...]
    agg = (acc_ref[0] + acc_ref[1]) * dinv
    h = jnp.dot(agg, w_ref[...], preferred_element_type=_F32,
                precision=lax.Precision.HIGHEST) + b_ref[...]
    hv = h[:N]
    mean = jnp.mean(hv, axis=0, keepdims=True)
    var = jnp.mean((hv - mean) ** 2, axis=0, keepdims=True)
    hn = (h - mean) * lax.rsqrt(var + 1e-5) * g_ref[...] + bt_ref[...]
    hs_ref[...] = jnp.maximum(hn, 0.0) * dinv


def _head_body(acc_ref, dinv_ref, w2_ref, b2_ref, w3_ref, b3_ref,
               f_ref, o_ref):
    agg = (acc_ref[0] + acc_ref[1]) * dinv_ref[...]
    f_ref[...] = jnp.dot(agg, w2_ref[...], preferred_element_type=_F32,
                         precision=lax.Precision.HIGHEST) + b2_ref[...]
    o_ref[...] = jnp.dot(agg, w3_ref[...], preferred_element_type=_F32,
                         precision=lax.Precision.HIGHEST) + b3_ref[...]


def kernel(x, edge_index, W1, b1, gamma, beta, W2, b2, W3, b3):
    row = edge_index[0]
    col = edge_index[1]
    pad_e = jnp.full((E_PAD - E,), N, jnp.int32)
    rowp = jnp.concatenate([row, pad_e])
    colp = jnp.concatenate([col, pad_e])
    x_pad = jnp.concatenate([x, jnp.zeros((N_PAD - N, D), _F32)], axis=0)

    deg_call = pl.kernel(
        _deg_body,
        out_type=jax.ShapeDtypeStruct((NC, N_PAD), _F32),
        mesh=_sc_mesh(),
        scratch_types=[
            pltpu.VMEM((K,), jnp.int32),
            pltpu.VMEM((K,), _F32),
            pltpu.VMEM((RPT,), _F32),
            pltpu.VMEM_SHARED((N_PAD,), _F32),
        ])
    agg_call = pl.kernel(
        _agg_body,
        out_type=jax.ShapeDtypeStruct((NC, N_PAD, D), _F32),
        mesh=_sc_mesh(),
        scratch_types=(
            [pltpu.VMEM((K,), jnp.int32) for _ in range(2 * NBUF)]
            + [pltpu.VMEM((K, D), _F32) for _ in range(NBUF)]
            + [pltpu.VMEM_SHARED((N_PAD, D), _F32)]
            + [pltpu.SemaphoreType.DMA for _ in range(NBUF)]
        ))

    scale_call = pl.pallas_call(
        _scale_body,
        out_shape=(jax.ShapeDtypeStruct((N_PAD, 1), _F32),
                   jax.ShapeDtypeStruct((N_PAD, D), _F32)))
    mid_call = pl.pallas_call(
        _mid_body,
        out_shape=jax.ShapeDtypeStruct((N_PAD, D), _F32))
    head_call = pl.pallas_call(
        _head_body,
        out_shape=(jax.ShapeDtypeStruct((N_PAD, D), _F32),
                   jax.ShapeDtypeStruct((N_PAD, D), _F32)))

    deg2 = deg_call(colp)
    deg_col = (deg2[0] + deg2[1]).reshape(N_PAD, 1)
    dinv_col, xs = scale_call(deg_col, x_pad)
    acc1 = agg_call(xs, rowp, colp)
    hs = mid_call(acc1, dinv_col, W1, b1.reshape(1, D), gamma.reshape(1, D),
                  beta.reshape(1, D))
    acc2 = agg_call(hs, rowp, colp)
    features, out = head_call(acc2, dinv_col, W2, b2.reshape(1, D), W3,
                              b3.reshape(1, D))
    return features[:N], out[:N]


# speedup vs baseline: 2.9679x; 2.1055x over previous
import jax
import jax.numpy as jnp
from jax import lax
from jax.experimental import pallas as pl
from jax.experimental.pallas import tpu as pltpu
from jax.experimental.pallas import tpu_sc as plsc

N = 10000
D = 128
E = 320000
NC = 2
NS = 16
NW = NC * NS
K = 128
N_PAD = 10240
E_PAD = 327680
RPT = N_PAD // NS
NBLK_FAST = 80
NBLK_SLOW = 80
FAST_CORE = 0
NBLK_DEG = (E_PAD // K) // NW
EPT = NBLK_FAST * K

_F32 = jnp.float32


def _sc_mesh():
    return plsc.VectorSubcoreMesh(
        core_axis_name="c", subcore_axis_name="s",
        num_cores=NC, num_subcores=NS)


def _deg_body(col_hbm, out_hbm, colv, onesv, zv, acc):
    c = lax.axis_index("c")
    s = lax.axis_index("s")
    wid = c * NS + s
    for j in range(K // 16):
        onesv[pl.ds(j * 16, 16)] = jnp.ones((16,), _F32)

    def zfill(j, carry):
        zv[pl.ds(j * 16, 16)] = jnp.zeros((16,), _F32)
        return carry

    lax.fori_loop(0, RPT // 16, zfill, 0)
    pltpu.sync_copy(zv, acc.at[pl.ds(s * RPT, RPT)])
    plsc.subcore_barrier()

    def step(i, carry):
        pltpu.sync_copy(col_hbm.at[pl.ds((wid * NBLK_DEG + i) * K, K)], colv)
        pltpu.sync_copy(onesv, acc.at[colv], add=True)
        return carry

    lax.fori_loop(0, NBLK_DEG, step, 0)
    plsc.subcore_barrier()
    pltpu.sync_copy(acc.at[pl.ds(s * RPT, RPT)],
                    out_hbm.at[c, pl.ds(s * RPT, RPT)])


NBUF = 2


def _agg_body(table_hbm, row_hbm, col_hbm, out_hbm, *refs):
    rowv = refs[0:NBUF]
    colv = refs[NBUF:2 * NBUF]
    msgs = refs[2 * NBUF:3 * NBUF]
    acc = refs[3 * NBUF]
    gsem = refs[3 * NBUF + 1:3 * NBUF + 1 + NBUF]
    c = lax.axis_index("c")
    s = lax.axis_index("s")

    fast = c == FAST_CORE
    base0 = jnp.where(fast, s * EPT, NS * EPT + s * NBLK_SLOW * K)
    base0 = pl.multiple_of(base0, 8)
    nblk = jnp.where(fast, NBLK_FAST, NBLK_SLOW)

    with jax.named_scope("agg_zero"):
        def zrow(i, carry):
            for j in range(D // 16):
                msgs[0][i, pl.ds(j * 16, 16)] = jnp.zeros((16,), _F32)
            return carry

        lax.fori_loop(0, K, zrow, 0)
        for b in range(RPT // K):
            pltpu.sync_copy(msgs[0], acc.at[pl.ds(s * RPT + b * K, K)])
        plsc.subcore_barrier()

    def fetch(p, j):
        base = base0 + j * K
        pltpu.sync_copy(row_hbm.at[pl.ds(base, K)], rowv[p])
        pltpu.sync_copy(col_hbm.at[pl.ds(base, K)], colv[p])
        pltpu.async_copy(table_hbm.at[rowv[p]], msgs[p], gsem[p])

    def drain_scatter(p):
        pltpu.make_async_copy(table_hbm.at[rowv[p]], msgs[p], gsem[p]).wait()
        pltpu.sync_copy(msgs[p], acc.at[colv[p]], add=True)

    with jax.named_scope("agg_edges"):
        for p in range(NBUF):
            fetch(p, p)

        def step(i, carry):
            j = i * NBUF
            for p in range(NBUF):
                drain_scatter(p)
                fetch(p, j + NBUF + p)
            return carry

        lax.fori_loop(0, nblk // NBUF - 1, step, 0)
        for p in range(NBUF):
            drain_scatter(p)
        plsc.subcore_barrier()
    with jax.named_scope("agg_writeout"):
        for b in range(RPT // K):
            r0 = s * RPT + b * K
            pltpu.sync_copy(acc.at[pl.ds(r0, K)], out_hbm.at[c, pl.ds(r0, K)])


def _scale_body(deg_ref, x_ref, dinv_ref, xs_ref):
    d = deg_ref[...]
    dinv = jnp.where(d > 0.0, lax.rsqrt(d), 0.0)
    dinv_ref[...] = dinv
    xs_ref[...] = x_ref[...] * dinv


def _mid_body(acc_ref, dinv_ref, w_ref, b_ref, g_ref, bt_ref, hs_ref):
    dinv = dinv_ref[...]
    agg = (acc_ref[0] + acc_ref[1]) * dinv
    h = jnp.dot(agg, w_ref[...], preferred_element_type=_F32,
                precision=lax.Precision.HIGHEST) + b_ref[...]
    hv = h[:N]
    mean = jnp.mean(hv, axis=0, keepdims=True)
    var = jnp.mean((hv - mean) ** 2, axis=0, keepdims=True)
    hn = (h - mean) * lax.rsqrt(var + 1e-5) * g_ref[...] + bt_ref[...]
    hs_ref[...] = jnp.maximum(hn, 0.0) * dinv


def _head_body(acc_ref, dinv_ref, w2_ref, b2_ref, w3_ref, b3_ref,
               f_ref, o_ref):
    agg = (acc_ref[0] + acc_ref[1]) * dinv_ref[...]
    f_ref[...] = jnp.dot(agg, w2_ref[...], preferred_element_type=_F32,
                         precision=lax.Precision.HIGHEST) + b2_ref[...]
    o_ref[...] = jnp.dot(agg, w3_ref[...], preferred_element_type=_F32,
                         precision=lax.Precision.HIGHEST) + b3_ref[...]


def kernel(x, edge_index, W1, b1, gamma, beta, W2, b2, W3, b3):
    row = edge_index[0]
    col = edge_index[1]
    pad_e = N + jnp.arange(E_PAD - E, dtype=jnp.int32) % (N_PAD - N)
    rowp = jnp.concatenate([row, pad_e])
    colp = jnp.concatenate([col, pad_e])
    x_pad = jnp.concatenate([x, jnp.zeros((N_PAD - N, D), _F32)], axis=0)

    deg_call = pl.kernel(
        _deg_body,
        out_type=jax.ShapeDtypeStruct((NC, N_PAD), _F32),
        mesh=_sc_mesh(),
        scratch_types=[
            pltpu.VMEM((K,), jnp.int32),
            pltpu.VMEM((K,), _F32),
            pltpu.VMEM((RPT,), _F32),
            pltpu.VMEM_SHARED((N_PAD,), _F32),
        ])
    agg_call = pl.kernel(
        _agg_body,
        out_type=jax.ShapeDtypeStruct((NC, N_PAD, D), _F32),
        mesh=_sc_mesh(),
        scratch_types=(
            [pltpu.VMEM((K,), jnp.int32) for _ in range(2 * NBUF)]
            + [pltpu.VMEM((K, D), _F32) for _ in range(NBUF)]
            + [pltpu.VMEM_SHARED((N_PAD, D), _F32)]
            + [pltpu.SemaphoreType.DMA for _ in range(NBUF)]
        ))

    scale_call = pl.pallas_call(
        _scale_body,
        out_shape=(jax.ShapeDtypeStruct((N_PAD, 1), _F32),
                   jax.ShapeDtypeStruct((N_PAD, D), _F32)))
    mid_call = pl.pallas_call(
        _mid_body,
        out_shape=jax.ShapeDtypeStruct((N_PAD, D), _F32))
    head_call = pl.pallas_call(
        _head_body,
        out_shape=(jax.ShapeDtypeStruct((N_PAD, D), _F32),
                   jax.ShapeDtypeStruct((N_PAD, D), _F32)))

    deg2 = deg_call(colp)
    deg_col = (deg2[0] + deg2[1]).reshape(N_PAD, 1)
    dinv_col, xs = scale_call(deg_col, x_pad)
    acc1 = agg_call(xs, rowp, colp)
    hs = mid_call(acc1, dinv_col, W1, b1.reshape(1, D), gamma.reshape(1, D),
                  beta.reshape(1, D))
    acc2 = agg_call(hs, rowp, colp)
    features, out = head_call(acc2, dinv_col, W2, b2.reshape(1, D), W3,
                              b3.reshape(1, D))
    return features[:N], out[:N]


# speedup vs baseline: 4.2118x; 1.4191x over previous
import jax
import jax.numpy as jnp
from jax import lax
from jax.experimental import pallas as pl
from jax.experimental.pallas import tpu as pltpu
from jax.experimental.pallas import tpu_sc as plsc

N = 10000
D = 128
E = 320000
NC = 2
NS = 16
NW = NC * NS
K = 128
N_PAD = 10240
E_PAD = 327680
RPT = N_PAD // NS
NBLK_FAST = 80
NBLK_SLOW = 80
FAST_CORE = 0
NBLK_DEG = (E_PAD // K) // NW
EPT = NBLK_FAST * K

_F32 = jnp.float32


def _sc_mesh():
    return plsc.VectorSubcoreMesh(
        core_axis_name="c", subcore_axis_name="s",
        num_cores=NC, num_subcores=NS)


IBUF = 4
MBUF = 2


def _deg_body(col_hbm, out_hbm, *refs):
    colv = refs[0:IBUF]
    onesv = refs[IBUF]
    zv = refs[IBUF + 1]
    acc = refs[IBUF + 2]
    isem = refs[IBUF + 3:IBUF + 3 + IBUF]
    c = lax.axis_index("c")
    s = lax.axis_index("s")
    wid = c * NS + s
    base0 = pl.multiple_of(wid * (NBLK_DEG * K), 8)

    def fetch_idx(q, j):
        pltpu.async_copy(col_hbm.at[pl.ds(base0 + j * K, K)], colv[q],
                         isem[q])

    def wait_idx(q):
        pltpu.make_async_copy(col_hbm.at[pl.ds(0, K)], colv[q],
                              isem[q]).wait()

    for q in range(IBUF):
        fetch_idx(q, q)
    for j in range(K // 16):
        onesv[pl.ds(j * 16, 16)] = jnp.ones((16,), _F32)

    def zfill(j, carry):
        zv[pl.ds(j * 16, 16)] = jnp.zeros((16,), _F32)
        return carry

    lax.fori_loop(0, RPT // 16, zfill, 0)
    pltpu.sync_copy(zv, acc.at[pl.ds(s * RPT, RPT)])
    plsc.subcore_barrier()

    def step(i, carry):
        j = i * IBUF
        for q in range(IBUF):
            wait_idx(q)
            pltpu.sync_copy(onesv, acc.at[colv[q]], add=True)
            fetch_idx(q, j + IBUF + q)
        return carry

    lax.fori_loop(0, NBLK_DEG // IBUF - 1, step, 0)
    for q in range(IBUF):
        wait_idx(q)
        pltpu.sync_copy(onesv, acc.at[colv[q]], add=True)
    plsc.subcore_barrier()
    pltpu.sync_copy(acc.at[pl.ds(s * RPT, RPT)],
                    out_hbm.at[c, pl.ds(s * RPT, RPT)])


NBUF = 2


def _agg_body(table_hbm, row_hbm, col_hbm, out_hbm, *refs):
    rowv = refs[0:IBUF]
    colv = refs[IBUF:2 * IBUF]
    msgs = refs[2 * IBUF:2 * IBUF + MBUF]
    acc = refs[2 * IBUF + MBUF]
    gsem = refs[2 * IBUF + MBUF + 1:2 * IBUF + MBUF + 1 + MBUF]
    isem = refs[2 * IBUF + MBUF + 1 + MBUF:]
    c = lax.axis_index("c")
    s = lax.axis_index("s")

    fast = c == FAST_CORE
    base0 = jnp.where(fast, s * EPT, NS * EPT + s * NBLK_SLOW * K)
    base0 = pl.multiple_of(base0, 8)
    nblk = jnp.where(fast, NBLK_FAST, NBLK_SLOW)

    def fetch_idx(q, j):
        base = base0 + j * K
        pltpu.async_copy(row_hbm.at[pl.ds(base, K)], rowv[q], isem[q])
        pltpu.async_copy(col_hbm.at[pl.ds(base, K)], colv[q], isem[q])

    def wait_idx(q):
        pltpu.make_async_copy(row_hbm.at[pl.ds(0, K)], rowv[q],
                              isem[q]).wait()
        pltpu.make_async_copy(col_hbm.at[pl.ds(0, K)], colv[q],
                              isem[q]).wait()

    def start_gather(p, q):
        pltpu.async_copy(table_hbm.at[rowv[q]], msgs[p], gsem[p])

    def drain_scatter(p, q):
        pltpu.make_async_copy(table_hbm.at[rowv[q]], msgs[p],
                              gsem[p]).wait()
        pltpu.sync_copy(msgs[p], acc.at[colv[q]], add=True)

    for q in range(IBUF):
        fetch_idx(q, q)

    with jax.named_scope("agg_zero"):
        def zrow(i, carry):
            for j in range(D // 16):
                msgs[0][i, pl.ds(j * 16, 16)] = jnp.zeros((16,), _F32)
            return carry

        lax.fori_loop(0, K, zrow, 0)
        for b in range(RPT // K):
            pltpu.sync_copy(msgs[0], acc.at[pl.ds(s * RPT + b * K, K)])
        plsc.subcore_barrier()

    with jax.named_scope("agg_edges"):
        wait_idx(0)
        start_gather(0, 0)
        wait_idx(1)
        start_gather(1, 1)

        def step(i, carry):
            j = i * IBUF
            for u in range(IBUF):
                p = u % MBUF
                drain_scatter(p, u)
                fetch_idx(u, j + IBUF + u)
                wait_idx((u + MBUF) % IBUF)
                start_gather(p, (u + MBUF) % IBUF)
            return carry

        lax.fori_loop(0, nblk // IBUF - 1, step, 0)
        for u in range(MBUF):
            drain_scatter(u % MBUF, u)
            wait_idx((u + MBUF) % IBUF)
            start_gather(u % MBUF, (u + MBUF) % IBUF)
        for u in range(MBUF, IBUF):
            drain_scatter(u % MBUF, u)
        plsc.subcore_barrier()
    with jax.named_scope("agg_writeout"):
        for b in range(RPT // K):
            r0 = s * RPT + b * K
            pltpu.sync_copy(acc.at[pl.ds(r0, K)], out_hbm.at[c, pl.ds(r0, K)])


def _scale_body(deg_ref, x_ref, dinv_ref, xs_ref):
    d = deg_ref[...]
    dinv = jnp.where(d > 0.0, lax.rsqrt(d), 0.0)
    dinv_ref[...] = dinv
    xs_ref[...] = x_ref[...] * dinv


def _mid_body(acc_ref, dinv_ref, w_ref, b_ref, g_ref, bt_ref, hs_ref):
    dinv = dinv_ref[...]
    agg = (acc_ref[0] + acc_ref[1]) * dinv
    h = jnp.dot(agg, w_ref[...], preferred_element_type=_F32,
                precision=lax.Precision.HIGHEST) + b_ref[...]
    hv = h[:N]
    mean = jnp.mean(hv, axis=0, keepdims=True)
    var = jnp.mean((hv - mean) ** 2, axis=0, keepdims=True)
    hn = (h - mean) * lax.rsqrt(var + 1e-5) * g_ref[...] + bt_ref[...]
    hs_ref[...] = jnp.maximum(hn, 0.0) * dinv


def _head_body(acc_ref, dinv_ref, w2_ref, b2_ref, w3_ref, b3_ref,
               f_ref, o_ref):
    agg = ((acc_ref[0, :N] + acc_ref[1, :N]) * dinv_ref[:N])
    f_ref[...] = jnp.dot(agg, w2_ref[...], preferred_element_type=_F32,
                         precision=lax.Precision.HIGHEST) + b2_ref[...]
    o_ref[...] = jnp.dot(agg, w3_ref[...], preferred_element_type=_F32,
                         precision=lax.Precision.HIGHEST) + b3_ref[...]


def kernel(x, edge_index, W1, b1, gamma, beta, W2, b2, W3, b3):
    row = edge_index[0]
    col = edge_index[1]
    pad_e = N + jnp.arange(E_PAD - E, dtype=jnp.int32) % (N_PAD - N)
    rowp = jnp.concatenate([row, pad_e])
    colp = jnp.concatenate([col, pad_e])
    x_pad = jnp.concatenate([x, jnp.zeros((N_PAD - N, D), _F32)], axis=0)

    deg_call = pl.kernel(
        _deg_body,
        out_type=jax.ShapeDtypeStruct((NC, N_PAD), _F32),
        mesh=_sc_mesh(),
        scratch_types=(
            [pltpu.VMEM((K,), jnp.int32) for _ in range(IBUF)]
            + [pltpu.VMEM((K,), _F32),
               pltpu.VMEM((RPT,), _F32),
               pltpu.VMEM_SHARED((N_PAD,), _F32)]
            + [pltpu.SemaphoreType.DMA for _ in range(IBUF)]
        ))
    agg_call = pl.kernel(
        _agg_body,
        out_type=jax.ShapeDtypeStruct((NC, N_PAD, D), _F32),
        mesh=_sc_mesh(),
        scratch_types=(
            [pltpu.VMEM((K,), jnp.int32) for _ in range(2 * IBUF)]
            + [pltpu.VMEM((K, D), _F32) for _ in range(MBUF)]
            + [pltpu.VMEM_SHARED((N_PAD, D), _F32)]
            + [pltpu.SemaphoreType.DMA for _ in range(MBUF + IBUF)]
        ))

    scale_call = pl.pallas_call(
        _scale_body,
        out_shape=(jax.ShapeDtypeStruct((N_PAD, 1), _F32),
                   jax.ShapeDtypeStruct((N_PAD, D), _F32)))
    mid_call = pl.pallas_call(
        _mid_body,
        out_shape=jax.ShapeDtypeStruct((N_PAD, D), _F32))
    head_call = pl.pallas_call(
        _head_body,
        out_shape=(jax.ShapeDtypeStruct((N, D), _F32),
                   jax.ShapeDtypeStruct((N, D), _F32)))

    deg2 = deg_call(colp)
    deg_col = (deg2[0] + deg2[1]).reshape(N_PAD, 1)
    dinv_col, xs = scale_call(deg_col, x_pad)
    acc1 = agg_call(xs, rowp, colp)
    hs = mid_call(acc1, dinv_col, W1, b1.reshape(1, D), gamma.reshape(1, D),
                  beta.reshape(1, D))
    acc2 = agg_call(hs, rowp, colp)
    features, out = head_call(acc2, dinv_col, W2, b2.reshape(1, D), W3,
                              b3.reshape(1, D))
    return features, out
